# Initial kernel scaffold; baseline (speedup 1.0000x reference)
import jax
import jax.numpy as jnp
from jax.experimental import pallas as pl


def kernel(x):
    raise NotImplementedError("write your pallas kernel here")


# speedup vs baseline: 116.8298x; 116.8298x over previous
import functools
import math

import ml_dtypes
import numpy as np
import jax
import jax.numpy as jnp
from jax import lax
from jax.experimental import pallas as pl
from jax.experimental.pallas import tpu as pltpu
from jax.experimental.pallas import tpu_sc as plsc

_BORDER = 16
_NUMF = 500
_MRSIZE = 3.0
_NLEVELS = 3
_INIT_SIGMA = 1.6
_NOCT = 3
_SIZES = (512, 256, 128)

_SEG_BASES = (0, 786432, 983040)
_TOTAL = 1032192
_NW = 32
_CAP = 128
_NSYN = 512
_NCAND = _NW * _CAP + _NSYN


def _sig_list():
    sig = [_INIT_SIGMA]
    sd = []
    for i in range(1, _NLEVELS + 2):
        s_tot = _INIT_SIGMA * (2.0 ** (float(i) / _NLEVELS))
        sd.append(math.sqrt(max(s_tot * s_tot - sig[-1] * sig[-1], 1e-8)))
        sig.append(s_tot)
    return sig, sd

_SIG, _SD = _sig_list()


def _gauss1d_np(sigma):
    r = max(int(math.ceil(3.0 * sigma)), 1)
    x = np.arange(-r, r + 1, dtype=np.float32)
    k = np.exp(-0.5 * (x / sigma) ** 2)
    return (k / k.sum()).astype(np.float32)


def _band_np(S, w):
    r = len(w) // 2
    M = np.zeros((S, S), np.float32)
    for k in range(len(w)):
        off = k - r
        i0 = max(0, -off)
        i1 = min(S, S - off)
        idx = np.arange(i0, i1)
        M[idx, idx + off] = w[k]
    return M


def _down_np(S):
    D = np.zeros((S // 2, S), np.float32)
    D[np.arange(S // 2), 2 * np.arange(S // 2)] = 1.0
    return D


def _build_mats():
    bf = ml_dtypes.bfloat16
    mats = []
    mats.append(_band_np(512, _gauss1d_np(_INIT_SIGMA)).astype(bf))
    for S in _SIZES:
        for i in range(4):
            mats.append(_band_np(S, _gauss1d_np(_SD[i])).astype(bf))
    mats.append(_down_np(512).astype(bf))
    mats.append(_down_np(512).T.copy().astype(bf))
    mats.append(_down_np(256).astype(bf))
    mats.append(_down_np(256).T.copy().astype(bf))
    return mats

_MATS = _build_mats()

_NEG = float("-inf")


def _shift_x(a, d, fill):
    H, W = a.shape
    col = jnp.full((H, 1), fill, a.dtype)
    if d == 1:
        return jnp.concatenate([a[:, 1:], col], axis=1)
    return jnp.concatenate([col, a[:, :-1]], axis=1)


def _shift_y(a, d, fill):
    H, W = a.shape
    row = jnp.full((1, W), fill, a.dtype)
    if d == 1:
        return jnp.concatenate([a[1:, :], row], axis=0)
    return jnp.concatenate([row, a[:-1, :]], axis=0)


def _hessian_resp(l, sigma):
    z = jnp.float32(0.0)
    rn = _shift_x(l, 1, z)
    ln = _shift_x(l, -1, z)
    dn = _shift_y(l, 1, z)
    up = _shift_y(l, -1, z)
    gxx = ln + rn - 2.0 * l
    gyy = up + dn - 2.0 * l
    h = _shift_x(l, -1, z) - _shift_x(l, 1, z)
    gxy = 0.25 * (_shift_y(h, -1, z) - _shift_y(h, 1, z))
    r = jnp.float32(sigma ** 4) * (gxx * gyy - gxy * gxy)
    return jnp.maximum(r, 0.0)


def _smax3(a):
    m = jnp.maximum(a, jnp.maximum(_shift_x(a, 1, _NEG), _shift_x(a, -1, _NEG)))
    return jnp.maximum(m, jnp.maximum(_shift_y(m, 1, _NEG), _shift_y(m, -1, _NEG)))


def _fold_chunks(v):
    H, W = v.shape
    m = v[:, :128]
    for k in range(1, W // 128):
        m = jnp.maximum(m, v[:, k * 128:(k + 1) * 128])
    m = jnp.maximum(m[:H // 2, :], m[H // 2:, :])
    m = jnp.maximum(m[:H // 4, :], m[H // 4:, :])
    return m


def _mm(a, b):
    return jnp.dot(a, b, preferred_element_type=jnp.float32)


def _blur_bf(lv, B):
    y = _mm(B, lv).astype(jnp.bfloat16)
    return _mm(y, B).astype(jnp.bfloat16)


def _k1_body(*refs):
    x = refs[0][...].astype(jnp.bfloat16)
    g_init = refs[1][...]
    gs = {S: [refs[2 + 4 * si + i][...] for i in range(4)]
          for si, S in enumerate(_SIZES)}
    d0, d0t, d1, d1t = (refs[14][...], refs[15][...], refs[16][...],
                        refs[17][...])
    out = {512: refs[18], 256: refs[19], 128: refs[20]}
    tout = refs[21]

    folds = []
    cur = _blur_bf(x, g_init)
    for o, S in enumerate(_SIZES):
        levels = [cur]
        for i in range(4):
            levels.append(_blur_bf(levels[-1], gs[S][i]))
        resps = [_hessian_resp(levels[i].astype(jnp.float32), _SIG[i])
                 for i in range(5)]
        smax = [_smax3(r) for r in resps]
        yy = lax.broadcasted_iota(jnp.int32, (S, S), 0)
        xx = lax.broadcasted_iota(jnp.int32, (S, S), 1)
        bm = ((yy >= _BORDER) & (yy < S - _BORDER) &
              (xx >= _BORDER) & (xx < S - _BORDER))
        for li in range(1, 4):
            mx = jnp.maximum(smax[li - 1], jnp.maximum(smax[li], smax[li + 1]))
            ism = (resps[li] >= mx) & (resps[li] > 0.0) & bm
            vals = jnp.where(ism, resps[li], 0.0)
            out[S][li - 1, :, :] = vals
            folds.append(_fold_chunks(vals))
        if o == 0:
            cur = _mm(d0, _mm(levels[_NLEVELS], d0t)).astype(jnp.bfloat16)
        elif o == 1:
            cur = _mm(d1, _mm(levels[_NLEVELS], d1t)).astype(jnp.bfloat16)

    cm = jnp.concatenate(folds, axis=0)
    cmi = lax.bitcast_convert_type(cm, jnp.int32)

    def bs(_, lohi):
        lo, hi = lohi
        mid = lo + (hi - lo + 1) // 2
        cnt = jnp.sum((cmi >= mid).astype(jnp.int32))
        ge = cnt >= _NUMF
        return (jnp.where(ge, mid, lo), jnp.where(ge, hi, mid - 1))

    lo, _hi = lax.fori_loop(0, 31, bs, (jnp.int32(0), jnp.int32(2**31 - 2)))
    t0 = lax.bitcast_convert_type(lo, jnp.float32)
    tout[:, :] = jnp.full((8, 128), 1.0, jnp.float32) * t0


def _stage1(x2d, interpret=False):
    out_shape = [
        jax.ShapeDtypeStruct((3, 512, 512), jnp.float32),
        jax.ShapeDtypeStruct((3, 256, 256), jnp.float32),
        jax.ShapeDtypeStruct((3, 128, 128), jnp.float32),
        jax.ShapeDtypeStruct((8, 128), jnp.float32),
    ]
    return pl.pallas_call(_k1_body, out_shape=out_shape,
                          interpret=interpret)(x2d, *_MATS)


_SEGS = ((0, 24576), (786432, 6144), (983040, 1536))


def _sc_body(r0, r1, r2, thr, ov, oi, buf, cv, ci, tv):
    wid = lax.axis_index("s") * 2 + lax.axis_index("c")
    pltpu.sync_copy(thr.at[pl.ds(0, 16)], tv)
    tsplat = tv[...]
    for j in range(_CAP // 16):
        cv[pl.ds(j * 16, 16)] = jnp.full((16,), -1.0, jnp.float32)
        ci[pl.ds(j * 16, 16)] = jnp.full((16,), 1 << 29, jnp.int32)

    refs = (r0, r1, r2)
    cnt = jnp.int32(0)
    for si in range(3):
        base, per = _SEGS[si]
        start = wid * per
        pltpu.sync_copy(refs[si].at[pl.ds(start, per)], buf.at[pl.ds(0, per)])

        def body(i, c, base=base, start=start):
            v = buf[pl.ds(i * 16, 16)]
            m = (v >= tsplat) & (v > 0.0)
            n = plsc.all_reduce_population_count(m)[0]

            def found(c2):
                cc = jnp.minimum(c2, _CAP - 16)
                plsc.store_compressed(cv.at[pl.ds(cc, 16)], v, mask=m)
                iv = lax.iota(jnp.int32, 16) + (base + start + i * 16)
                plsc.store_compressed(ci.at[pl.ds(cc, 16)], iv, mask=m)
                return c2 + n

            return lax.cond(n > 0, found, lambda c2: c2, c)

        cnt = lax.fori_loop(0, per // 16, body, cnt)

    pltpu.sync_copy(cv, ov.at[wid])
    pltpu.sync_copy(ci, oi.at[wid])


def _stage2(r0f, r1f, r2f, thrf, interpret=False):
    mesh = plsc.VectorSubcoreMesh(core_axis_name="c", subcore_axis_name="s")
    f = pl.kernel(
        _sc_body,
        out_type=[
            jax.ShapeDtypeStruct((_NW, _CAP), jnp.float32),
            jax.ShapeDtypeStruct((_NW, _CAP), jnp.int32),
        ],
        mesh=mesh,
        scratch_types=[
            pltpu.VMEM((24576,), jnp.float32),
            pltpu.VMEM((_CAP,), jnp.float32),
            pltpu.VMEM((_CAP,), jnp.int32),
            pltpu.VMEM((16,), jnp.float32),
        ],
        compiler_params=pltpu.CompilerParams(needs_layout_passes=False),
        interpret=interpret,
    )
    return f(r0f, r1f, r2f, thrf)


_SIGLI = tuple(float(s) for s in _SIG[1:4])
_STAB = tuple(tuple(float(np.float32(_SIG[li] * (2.0 ** o) * _MRSIZE))
                    for li in (1, 2, 3)) for o in range(3))


def _k3_body(vcol_ref, icol_ref, vrow_ref, irow_ref,
             oval_ref, os_ref, ocx_ref, ocy_ref):
    vcol = vcol_ref[...]
    icol = icol_ref[...]
    rank_chunks = []
    for jb in range(_NCAND // 128):
        vj = vrow_ref[:, pl.ds(jb * 128, 128)]
        ij = irow_ref[:, pl.ds(jb * 128, 128)]
        beats = (vcol > vj) | ((vcol == vj) & (icol < ij))
        rank_chunks.append(jnp.sum(beats.astype(jnp.int32), axis=0,
                                   keepdims=True))
    ranks = jnp.concatenate(rank_chunks, axis=1)

    vrow = vrow_ref[...]
    irow = irow_ref[...]
    piota = lax.broadcasted_iota(jnp.int32, (512, _NCAND), 0)
    sel = ranks == piota
    val = jnp.sum(jnp.where(sel, vrow, 0.0), axis=1)
    idx = jnp.sum(jnp.where(sel, irow, 0), axis=1)

    is2 = idx >= _SEG_BASES[2]
    is1 = (idx >= _SEG_BASES[1]) & (~is2)
    q = idx - jnp.where(is2, _SEG_BASES[2],
                        jnp.where(is1, _SEG_BASES[1], 0))
    sh_li = jnp.where(is2, 14, jnp.where(is1, 16, 18))
    lix = lax.shift_right_logical(q, sh_li)
    rem = q & (lax.shift_left(jnp.ones_like(q), sh_li) - 1)
    sh_w = jnp.where(is2, 7, jnp.where(is1, 8, 9))
    y = lax.shift_right_logical(rem, sh_w)
    xq = rem & (lax.shift_left(jnp.ones_like(rem), sh_w) - 1)
    scale = jnp.where(is2, 4.0, jnp.where(is1, 2.0, 1.0)).astype(jnp.float32)

    def stab_sel(o):
        return jnp.where(lix == 0, _STAB[o][0],
                         jnp.where(lix == 1, _STAB[o][1], _STAB[o][2]))

    sv = jnp.where(is2, stab_sel(2), jnp.where(is1, stab_sel(1), stab_sel(0)))
    oval_ref[...] = val
    os_ref[...] = sv.astype(jnp.float32)
    ocx_ref[...] = xq.astype(jnp.float32) * scale
    ocy_ref[...] = y.astype(jnp.float32) * scale


def _stage3(vcol, icol, vrow, irow, interpret=False):
    out_shape = [jax.ShapeDtypeStruct((512,), jnp.float32)] * 4
    return pl.pallas_call(_k3_body, out_shape=out_shape,
                          interpret=interpret)(vcol, icol, vrow, irow)


def _run(x, interpret=False):
    x2d = x.reshape(512, 512)
    r0, r1, r2, t = _stage1(x2d, interpret=interpret)
    cv, ci = _stage2(r0.reshape(-1), r1.reshape(-1), r2.reshape(-1),
                     t.reshape(-1), interpret=interpret)
    vflat = jnp.concatenate([cv.reshape(-1), jnp.zeros((_NSYN,), jnp.float32)])
    iflat = jnp.concatenate([ci.reshape(-1),
                             jnp.arange(_NSYN, dtype=jnp.int32)])
    val, s, cx, cy = _stage3(vflat.reshape(_NCAND, 1),
                             iflat.reshape(_NCAND, 1),
                             vflat.reshape(1, _NCAND),
                             iflat.reshape(1, _NCAND),
                             interpret=interpret)
    top = val[:_NUMF]
    z = jnp.zeros((_NUMF,), jnp.float32)
    row0 = jnp.stack([s[:_NUMF], z, cx[:_NUMF]], axis=1)
    row1 = jnp.stack([z, s[:_NUMF], cy[:_NUMF]], axis=1)
    lafs = jnp.stack([row0, row1], axis=1)
    return top, lafs


def kernel(x):
    return _run(x)


# speedup vs baseline: 130.3624x; 1.1158x over previous
import functools
import math

import ml_dtypes
import numpy as np
import jax
import jax.numpy as jnp
from jax import lax
from jax.experimental import pallas as pl
from jax.experimental.pallas import tpu as pltpu
from jax.experimental.pallas import tpu_sc as plsc

_BORDER = 16
_NUMF = 500
_MRSIZE = 3.0
_NLEVELS = 3
_INIT_SIGMA = 1.6
_NOCT = 3
_SIZES = (512, 256, 128)

_SEG_BASES = (0, 786432, 983040)
_TOTAL = 1032192
_NW = 32
_CAP = 128
_NSYN = 512
_NCAND = _NW * _CAP + _NSYN


def _sig_list():
    sig = [_INIT_SIGMA]
    sd = []
    for i in range(1, _NLEVELS + 2):
        s_tot = _INIT_SIGMA * (2.0 ** (float(i) / _NLEVELS))
        sd.append(math.sqrt(max(s_tot * s_tot - sig[-1] * sig[-1], 1e-8)))
        sig.append(s_tot)
    return sig, sd

_SIG, _SD = _sig_list()


def _gauss1d_np(sigma):
    r = max(int(math.ceil(3.0 * sigma)), 1)
    x = np.arange(-r, r + 1, dtype=np.float32)
    k = np.exp(-0.5 * (x / sigma) ** 2)
    return (k / k.sum()).astype(np.float32)


def _band_np(S, w):
    r = len(w) // 2
    M = np.zeros((S, S), np.float32)
    for k in range(len(w)):
        off = k - r
        i0 = max(0, -off)
        i1 = min(S, S - off)
        idx = np.arange(i0, i1)
        M[idx, idx + off] = w[k]
    return M


def _down_np(S):
    D = np.zeros((S // 2, S), np.float32)
    D[np.arange(S // 2), 2 * np.arange(S // 2)] = 1.0
    return D


def _build_mats():
    bf = ml_dtypes.bfloat16
    mats = []
    mats.append(_band_np(512, _gauss1d_np(_INIT_SIGMA)).astype(bf))
    for S in _SIZES:
        for i in range(4):
            mats.append(_band_np(S, _gauss1d_np(_SD[i])).astype(bf))
    mats.append(_down_np(512).astype(bf))
    mats.append(_down_np(512).T.copy().astype(bf))
    mats.append(_down_np(256).astype(bf))
    mats.append(_down_np(256).T.copy().astype(bf))
    return mats

_MATS = _build_mats()

_NEG = float("-inf")


def _shift_x(a, d, fill):
    H, W = a.shape
    col = jnp.full((H, 1), fill, a.dtype)
    if d == 1:
        return jnp.concatenate([a[:, 1:], col], axis=1)
    return jnp.concatenate([col, a[:, :-1]], axis=1)


def _shift_y(a, d, fill):
    H, W = a.shape
    row = jnp.full((1, W), fill, a.dtype)
    if d == 1:
        return jnp.concatenate([a[1:, :], row], axis=0)
    return jnp.concatenate([row, a[:-1, :]], axis=0)


def _hessian_resp(l, sigma):
    z = jnp.float32(0.0)
    rn = _shift_x(l, 1, z)
    ln = _shift_x(l, -1, z)
    dn = _shift_y(l, 1, z)
    up = _shift_y(l, -1, z)
    gxx = ln + rn - 2.0 * l
    gyy = up + dn - 2.0 * l
    h = _shift_x(l, -1, z) - _shift_x(l, 1, z)
    gxy = 0.25 * (_shift_y(h, -1, z) - _shift_y(h, 1, z))
    r = jnp.float32(sigma ** 4) * (gxx * gyy - gxy * gxy)
    return jnp.maximum(r, 0.0)


def _smax3(a):
    m = jnp.maximum(a, jnp.maximum(_shift_x(a, 1, _NEG), _shift_x(a, -1, _NEG)))
    return jnp.maximum(m, jnp.maximum(_shift_y(m, 1, _NEG), _shift_y(m, -1, _NEG)))


def _fold_chunks(v):
    H, W = v.shape
    m = v[:, :128]
    for k in range(1, W // 128):
        m = jnp.maximum(m, v[:, k * 128:(k + 1) * 128])
    m = jnp.maximum(m[:H // 2, :], m[H // 2:, :])
    m = jnp.maximum(m[:H // 4, :], m[H // 4:, :])
    return m


def _mm(a, b):
    return jnp.dot(a, b, preferred_element_type=jnp.float32)


def _blur_bf(lv, B):
    y = _mm(B, lv).astype(jnp.bfloat16)
    return _mm(y, B).astype(jnp.bfloat16)


def _k1_body(*refs):
    x = refs[0][...].astype(jnp.bfloat16)
    g_init = refs[1][...]
    gs = {S: [refs[2 + 4 * si + i][...] for i in range(4)]
          for si, S in enumerate(_SIZES)}
    d0, d0t, d1, d1t = (refs[14][...], refs[15][...], refs[16][...],
                        refs[17][...])
    out = {512: refs[18], 256: refs[19], 128: refs[20]}
    tout = refs[21]
    fout = refs[22]

    folds = []
    rowmaxes = []
    cur = _blur_bf(x, g_init)
    for o, S in enumerate(_SIZES):
        levels = [cur]
        for i in range(4):
            levels.append(_blur_bf(levels[-1], gs[S][i]))
        resps = [_hessian_resp(levels[i].astype(jnp.float32), _SIG[i])
                 for i in range(5)]
        smax = [_smax3(r) for r in resps]
        yy = lax.broadcasted_iota(jnp.int32, (S, S), 0)
        xx = lax.broadcasted_iota(jnp.int32, (S, S), 1)
        bm = ((yy >= _BORDER) & (yy < S - _BORDER) &
              (xx >= _BORDER) & (xx < S - _BORDER))
        for li in range(1, 4):
            mx = jnp.maximum(smax[li - 1], jnp.maximum(smax[li], smax[li + 1]))
            ism = (resps[li] >= mx) & (resps[li] > 0.0) & bm
            vals = jnp.where(ism, resps[li], 0.0)
            out[S][li - 1, :, :] = vals
            folds.append(_fold_chunks(vals))
            rowmaxes.append(jnp.max(vals, axis=1, keepdims=True))
        if o == 0:
            cur = _mm(d0, _mm(levels[_NLEVELS], d0t)).astype(jnp.bfloat16)
        elif o == 1:
            cur = _mm(d1, _mm(levels[_NLEVELS], d1t)).astype(jnp.bfloat16)

    cm = jnp.concatenate(folds, axis=0)
    cmi = lax.bitcast_convert_type(cm, jnp.int32)

    def bs(_, lohi):
        lo, hi = lohi
        mid = lo + (hi - lo + 1) // 2
        cnt = jnp.sum((cmi >= mid).astype(jnp.int32))
        ge = cnt >= _NUMF
        return (jnp.where(ge, mid, lo), jnp.where(ge, hi, mid - 1))

    lo, _hi = lax.fori_loop(0, 31, bs, (jnp.int32(0), jnp.int32(2**31 - 2)))
    t0 = lax.bitcast_convert_type(lo, jnp.float32)
    tout[:, :] = jnp.full((8, 128), 1.0, jnp.float32) * t0
    rm = jnp.concatenate(rowmaxes, axis=0)
    fout[:, :] = ((rm >= t0) & (rm > 0.0)).astype(jnp.int32)


def _stage1(x2d, interpret=False):
    out_shape = [
        jax.ShapeDtypeStruct((3, 512, 512), jnp.float32),
        jax.ShapeDtypeStruct((3, 256, 256), jnp.float32),
        jax.ShapeDtypeStruct((3, 128, 128), jnp.float32),
        jax.ShapeDtypeStruct((8, 128), jnp.float32),
        jax.ShapeDtypeStruct((2688, 1), jnp.int32),
    ]
    return pl.pallas_call(_k1_body, out_shape=out_shape,
                          interpret=interpret)(x2d, *_MATS)


_SEGS = ((0, 24576), (786432, 6144), (983040, 1536))


_ROWSEG = ((0, 48, 512, 32, 0), (1536, 24, 256, 16, 786432),
           (2304, 12, 128, 8, 983040))


def _sc_body(r0, r1, r2, thr, flg, ov, oi, buf, cv, ci, tv, fb):
    wid = lax.axis_index("s") * 2 + lax.axis_index("c")
    pltpu.sync_copy(thr.at[pl.ds(0, 16)], tv)
    pltpu.sync_copy(flg, fb.at[pl.ds(0, 2688)])
    tsplat = tv[...]
    for j in range(_CAP // 16):
        cv[pl.ds(j * 16, 16)] = jnp.full((16,), -1.0, jnp.float32)
        ci[pl.ds(j * 16, 16)] = jnp.full((16,), 1 << 29, jnp.int32)

    refs = (r0, r1, r2)
    cnt = jnp.int32(0)
    for si in range(3):
        fbase, nrows, W, vpr, base = _ROWSEG[si]
        per = nrows * W
        start = wid * per
        pltpu.sync_copy(refs[si].at[pl.ds(start, per)], buf.at[pl.ds(0, per)])

        def rowbody(r, c, fbase=fbase, nrows=nrows, vpr=vpr,
                    base=base, start=start):
            flag = fb[pl.ds(fbase + wid * nrows + r, 16)][0]

            def scan_row(c2):
                def body(j, c3):
                    v = buf[pl.ds((r * vpr + j) * 16, 16)]
                    m = (v >= tsplat) & (v > 0.0)
                    n = plsc.all_reduce_population_count(m)[0]

                    def found(c4):
                        cc = jnp.minimum(c4, _CAP - 16)
                        plsc.store_compressed(cv.at[pl.ds(cc, 16)], v, mask=m)
                        iv = (lax.iota(jnp.int32, 16)
                              + (base + start + (r * vpr + j) * 16))
                        plsc.store_compressed(ci.at[pl.ds(cc, 16)], iv,
                                              mask=m)
                        return c4 + n

                    return lax.cond(n > 0, found, lambda c4: c4, c3)

                return lax.fori_loop(0, vpr, body, c2)

            return lax.cond(flag > 0, scan_row, lambda c2: c2, c)

        cnt = lax.fori_loop(0, nrows, rowbody, cnt)

    pltpu.sync_copy(cv, ov.at[wid])
    pltpu.sync_copy(ci, oi.at[wid])


def _stage2(r0f, r1f, r2f, thrf, flgf, interpret=False):
    mesh = plsc.VectorSubcoreMesh(core_axis_name="c", subcore_axis_name="s")
    f = pl.kernel(
        _sc_body,
        out_type=[
            jax.ShapeDtypeStruct((_NW, _CAP), jnp.float32),
            jax.ShapeDtypeStruct((_NW, _CAP), jnp.int32),
        ],
        mesh=mesh,
        scratch_types=[
            pltpu.VMEM((24576,), jnp.float32),
            pltpu.VMEM((_CAP,), jnp.float32),
            pltpu.VMEM((_CAP,), jnp.int32),
            pltpu.VMEM((16,), jnp.float32),
            pltpu.VMEM((2704,), jnp.int32),
        ],
        compiler_params=pltpu.CompilerParams(needs_layout_passes=False),
        interpret=interpret,
    )
    return f(r0f, r1f, r2f, thrf, flgf)


_SIGLI = tuple(float(s) for s in _SIG[1:4])
_STAB = tuple(tuple(float(np.float32(_SIG[li] * (2.0 ** o) * _MRSIZE))
                    for li in (1, 2, 3)) for o in range(3))


def _k3_body(vcol_ref, icol_ref, vrow_ref, irow_ref,
             oval_ref, os_ref, ocx_ref, ocy_ref):
    vcol = vcol_ref[...]
    icol = icol_ref[...]
    rank_chunks = []
    for jb in range(_NCAND // 128):
        vj = vrow_ref[:, pl.ds(jb * 128, 128)]
        ij = irow_ref[:, pl.ds(jb * 128, 128)]
        beats = (vcol > vj) | ((vcol == vj) & (icol < ij))
        rank_chunks.append(jnp.sum(beats.astype(jnp.int32), axis=0,
                                   keepdims=True))
    ranks = jnp.concatenate(rank_chunks, axis=1)

    vrow = vrow_ref[...]
    irow = irow_ref[...]
    piota = lax.broadcasted_iota(jnp.int32, (512, _NCAND), 0)
    sel = ranks == piota
    val = jnp.sum(jnp.where(sel, vrow, 0.0), axis=1)
    idx = jnp.sum(jnp.where(sel, irow, 0), axis=1)

    is2 = idx >= _SEG_BASES[2]
    is1 = (idx >= _SEG_BASES[1]) & (~is2)
    q = idx - jnp.where(is2, _SEG_BASES[2],
                        jnp.where(is1, _SEG_BASES[1], 0))
    sh_li = jnp.where(is2, 14, jnp.where(is1, 16, 18))
    lix = lax.shift_right_logical(q, sh_li)
    rem = q & (lax.shift_left(jnp.ones_like(q), sh_li) - 1)
    sh_w = jnp.where(is2, 7, jnp.where(is1, 8, 9))
    y = lax.shift_right_logical(rem, sh_w)
    xq = rem & (lax.shift_left(jnp.ones_like(rem), sh_w) - 1)
    scale = jnp.where(is2, 4.0, jnp.where(is1, 2.0, 1.0)).astype(jnp.float32)

    def stab_sel(o):
        return jnp.where(lix == 0, _STAB[o][0],
                         jnp.where(lix == 1, _STAB[o][1], _STAB[o][2]))

    sv = jnp.where(is2, stab_sel(2), jnp.where(is1, stab_sel(1), stab_sel(0)))
    oval_ref[...] = val
    os_ref[...] = sv.astype(jnp.float32)
    ocx_ref[...] = xq.astype(jnp.float32) * scale
    ocy_ref[...] = y.astype(jnp.float32) * scale


def _stage3(vcol, icol, vrow, irow, interpret=False):
    out_shape = [jax.ShapeDtypeStruct((512,), jnp.float32)] * 4
    return pl.pallas_call(_k3_body, out_shape=out_shape,
                          interpret=interpret)(vcol, icol, vrow, irow)


def _run(x, interpret=False):
    x2d = x.reshape(512, 512)
    r0, r1, r2, t, flg = _stage1(x2d, interpret=interpret)
    cv, ci = _stage2(r0.reshape(-1), r1.reshape(-1), r2.reshape(-1),
                     t.reshape(-1), flg.reshape(-1), interpret=interpret)
    vflat = jnp.concatenate([cv.reshape(-1), jnp.zeros((_NSYN,), jnp.float32)])
    iflat = jnp.concatenate([ci.reshape(-1),
                             jnp.arange(_NSYN, dtype=jnp.int32)])
    val, s, cx, cy = _stage3(vflat.reshape(_NCAND, 1),
                             iflat.reshape(_NCAND, 1),
                             vflat.reshape(1, _NCAND),
                             iflat.reshape(1, _NCAND),
                             interpret=interpret)
    top = val[:_NUMF]
    z = jnp.zeros((_NUMF,), jnp.float32)
    row0 = jnp.stack([s[:_NUMF], z, cx[:_NUMF]], axis=1)
    row1 = jnp.stack([z, s[:_NUMF], cy[:_NUMF]], axis=1)
    lafs = jnp.stack([row0, row1], axis=1)
    return top, lafs


def kernel(x):
    return _run(x)


# speedup vs baseline: 163.7088x; 1.2558x over previous
import functools
import math

import ml_dtypes
import numpy as np
import jax
import jax.numpy as jnp
from jax import lax
from jax.experimental import pallas as pl
from jax.experimental.pallas import tpu as pltpu
from jax.experimental.pallas import tpu_sc as plsc

_BORDER = 16
_NUMF = 500
_MRSIZE = 3.0
_NLEVELS = 3
_INIT_SIGMA = 1.6
_NOCT = 3
_SIZES = (512, 256, 128)

_SEG_BASES = (0, 786432, 983040)
_TOTAL = 1032192
_NW = 32
_CAP = 128
_CCAP = 1024
_NSYN = 512
_NCAND = 2 * _CCAP + _NSYN


def _sig_list():
    sig = [_INIT_SIGMA]
    sd = []
    for i in range(1, _NLEVELS + 2):
        s_tot = _INIT_SIGMA * (2.0 ** (float(i) / _NLEVELS))
        sd.append(math.sqrt(max(s_tot * s_tot - sig[-1] * sig[-1], 1e-8)))
        sig.append(s_tot)
    return sig, sd

_SIG, _SD = _sig_list()


def _gauss1d_np(sigma):
    r = max(int(math.ceil(3.0 * sigma)), 1)
    x = np.arange(-r, r + 1, dtype=np.float32)
    k = np.exp(-0.5 * (x / sigma) ** 2)
    return (k / k.sum()).astype(np.float32)


def _band_np(S, w):
    r = len(w) // 2
    M = np.zeros((S, S), np.float32)
    for k in range(len(w)):
        off = k - r
        i0 = max(0, -off)
        i1 = min(S, S - off)
        idx = np.arange(i0, i1)
        M[idx, idx + off] = w[k]
    return M


def _down_np(S):
    D = np.zeros((S // 2, S), np.float32)
    D[np.arange(S // 2), 2 * np.arange(S // 2)] = 1.0
    return D


def _build_mats():
    bf = ml_dtypes.bfloat16
    mats = []
    mats.append(_band_np(512, _gauss1d_np(_INIT_SIGMA)).astype(bf))
    for S in _SIZES:
        for i in range(4):
            mats.append(_band_np(S, _gauss1d_np(_SD[i])).astype(bf))
    mats.append(_down_np(512).astype(bf))
    mats.append(_down_np(512).T.copy().astype(bf))
    mats.append(_down_np(256).astype(bf))
    mats.append(_down_np(256).T.copy().astype(bf))
    return mats

_MATS = _build_mats()

_NEG = float("-inf")


def _shift_x(a, d, fill):
    H, W = a.shape
    col = jnp.full((H, 1), fill, a.dtype)
    if d == 1:
        return jnp.concatenate([a[:, 1:], col], axis=1)
    return jnp.concatenate([col, a[:, :-1]], axis=1)


def _shift_y(a, d, fill):
    H, W = a.shape
    row = jnp.full((1, W), fill, a.dtype)
    if d == 1:
        return jnp.concatenate([a[1:, :], row], axis=0)
    return jnp.concatenate([row, a[:-1, :]], axis=0)


def _hessian_resp(l, sigma):
    z = jnp.float32(0.0)
    rn = _shift_x(l, 1, z)
    ln = _shift_x(l, -1, z)
    dn = _shift_y(l, 1, z)
    up = _shift_y(l, -1, z)
    gxx = ln + rn - 2.0 * l
    gyy = up + dn - 2.0 * l
    h = _shift_x(l, -1, z) - _shift_x(l, 1, z)
    gxy = 0.25 * (_shift_y(h, -1, z) - _shift_y(h, 1, z))
    r = jnp.float32(sigma ** 4) * (gxx * gyy - gxy * gxy)
    return jnp.maximum(r, 0.0)


def _smax3(a):
    m = jnp.maximum(a, jnp.maximum(_shift_x(a, 1, _NEG), _shift_x(a, -1, _NEG)))
    return jnp.maximum(m, jnp.maximum(_shift_y(m, 1, _NEG), _shift_y(m, -1, _NEG)))


def _fold_chunks(v):
    H, W = v.shape
    m = v[:, :128]
    for k in range(1, W // 128):
        m = jnp.maximum(m, v[:, k * 128:(k + 1) * 128])
    m = jnp.maximum(m[:H // 2, :], m[H // 2:, :])
    m = jnp.maximum(m[:H // 4, :], m[H // 4:, :])
    return m


def _mm(a, b):
    return jnp.dot(a, b, preferred_element_type=jnp.float32)


def _blur_bf(lv, B):
    y = _mm(B, lv).astype(jnp.bfloat16)
    return _mm(y, B).astype(jnp.bfloat16)


def _k1_body(*refs):
    x = refs[0][...].astype(jnp.bfloat16)
    g_init = refs[1][...]
    gs = {S: [refs[2 + 4 * si + i][...] for i in range(4)]
          for si, S in enumerate(_SIZES)}
    d0, d0t, d1, d1t = (refs[14][...], refs[15][...], refs[16][...],
                        refs[17][...])
    out = {512: refs[18], 256: refs[19], 128: refs[20]}
    tout = refs[21]
    fout = refs[22]

    folds = []
    rowmaxes = []
    cur = _blur_bf(x, g_init)
    for o, S in enumerate(_SIZES):
        levels = [cur]
        for i in range(4):
            levels.append(_blur_bf(levels[-1], gs[S][i]))
        resps = [_hessian_resp(levels[i].astype(jnp.float32), _SIG[i])
                 for i in range(5)]
        smax = [_smax3(r) for r in resps]
        yy = lax.broadcasted_iota(jnp.int32, (S, S), 0)
        xx = lax.broadcasted_iota(jnp.int32, (S, S), 1)
        bm = ((yy >= _BORDER) & (yy < S - _BORDER) &
              (xx >= _BORDER) & (xx < S - _BORDER))
        for li in range(1, 4):
            mx = jnp.maximum(smax[li - 1], jnp.maximum(smax[li], smax[li + 1]))
            ism = (resps[li] >= mx) & (resps[li] > 0.0) & bm
            vals = jnp.where(ism, resps[li], 0.0)
            out[S][li - 1, :, :] = vals
            folds.append(_fold_chunks(vals))
            rowmaxes.append(jnp.max(vals, axis=1, keepdims=True))
        if o == 0:
            cur = _mm(d0, _mm(levels[_NLEVELS], d0t)).astype(jnp.bfloat16)
        elif o == 1:
            cur = _mm(d1, _mm(levels[_NLEVELS], d1t)).astype(jnp.bfloat16)

    cm = jnp.concatenate(folds, axis=0)
    cmi = lax.bitcast_convert_type(cm, jnp.int32)

    def bs(_, lohi):
        lo, hi = lohi
        mid = lo + (hi - lo + 1) // 2
        cnt = jnp.sum((cmi >= mid).astype(jnp.int32))
        ge = cnt >= _NUMF
        return (jnp.where(ge, mid, lo), jnp.where(ge, hi, mid - 1))

    lo, _hi = lax.fori_loop(0, 31, bs, (jnp.int32(0), jnp.int32(2**31 - 2)))
    t0 = lax.bitcast_convert_type(lo, jnp.float32)
    tout[:, :] = jnp.full((8, 128), 1.0, jnp.float32) * t0
    rm = jnp.concatenate(rowmaxes, axis=0)
    fout[:, :] = ((rm >= t0) & (rm > 0.0)).astype(jnp.int32)


def _stage1(x2d, interpret=False):
    out_shape = [
        jax.ShapeDtypeStruct((3, 512, 512), jnp.float32),
        jax.ShapeDtypeStruct((3, 256, 256), jnp.float32),
        jax.ShapeDtypeStruct((3, 128, 128), jnp.float32),
        jax.ShapeDtypeStruct((8, 128), jnp.float32),
        jax.ShapeDtypeStruct((2688, 1), jnp.int32),
    ]
    return pl.pallas_call(_k1_body, out_shape=out_shape,
                          interpret=interpret)(x2d, *_MATS)


_SEGS = ((0, 24576), (786432, 6144), (983040, 1536))


_ROWSEG = ((0, 48, 512, 32, 0), (1536, 24, 256, 16, 786432),
           (2304, 12, 128, 8, 983040))


def _sc_body(r0, r1, r2, thr, flg, ov, oi, buf, cv, ci, tv, fb,
             sh_v, sh_i, lv, li, ob, ib):
    sid = lax.axis_index("s")
    cid = lax.axis_index("c")
    wid = sid * 2 + cid
    pltpu.sync_copy(thr.at[pl.ds(0, 16)], tv)
    pltpu.sync_copy(flg, fb.at[pl.ds(0, 2688)])
    tsplat = tv[...]
    for j in range(_CAP // 16):
        cv[pl.ds(j * 16, 16)] = jnp.full((16,), -1.0, jnp.float32)
        ci[pl.ds(j * 16, 16)] = jnp.full((16,), 1 << 29, jnp.int32)

    refs = (r0, r1, r2)
    cnt = jnp.int32(0)
    for si in range(3):
        fbase, nrows, W, vpr, base = _ROWSEG[si]
        per = nrows * W
        start = wid * per
        pltpu.sync_copy(refs[si].at[pl.ds(start, per)], buf.at[pl.ds(0, per)])

        def rowbody(r, c, fbase=fbase, nrows=nrows, vpr=vpr,
                    base=base, start=start):
            flag = fb[pl.ds(fbase + wid * nrows + r, 16)][0]

            def scan_row(c2):
                def body(j, c3):
                    v = buf[pl.ds((r * vpr + j) * 16, 16)]
                    m = (v >= tsplat) & (v > 0.0)
                    n = plsc.all_reduce_population_count(m)[0]

                    def found(c4):
                        cc = jnp.minimum(c4, _CAP - 16)
                        plsc.store_compressed(cv.at[pl.ds(cc, 16)], v, mask=m)
                        iv = (lax.iota(jnp.int32, 16)
                              + (base + start + (r * vpr + j) * 16))
                        plsc.store_compressed(ci.at[pl.ds(cc, 16)], iv,
                                              mask=m)
                        return c4 + n

                    return lax.cond(n > 0, found, lambda c4: c4, c3)

                return lax.fori_loop(0, vpr, body, c2)

            return lax.cond(flag > 0, scan_row, lambda c2: c2, c)

        cnt = lax.fori_loop(0, nrows, rowbody, cnt)

    pltpu.sync_copy(cv, sh_v.at[sid])
    pltpu.sync_copy(ci, sh_i.at[sid])
    plsc.subcore_barrier()

    @pl.when(sid == 0)
    def _compact():
        pltpu.sync_copy(sh_v, lv)
        pltpu.sync_copy(sh_i, li)
        for j in range(_CCAP // 16):
            ob[pl.ds(j * 16, 16)] = jnp.full((16,), -1.0, jnp.float32)
            ib[pl.ds(j * 16, 16)] = jnp.full((16,), 1 << 29, jnp.int32)

        def cbody(k, c):
            w = k // (_CAP // 16)
            j = k % (_CAP // 16)
            v = lv[w, pl.ds(j * 16, 16)]
            m = v > 0.0
            n = plsc.all_reduce_population_count(m)[0]

            def found(c2):
                cc = jnp.minimum(c2, _CCAP - 16)
                plsc.store_compressed(ob.at[pl.ds(cc, 16)], v, mask=m)
                iv = li[w, pl.ds(j * 16, 16)]
                plsc.store_compressed(ib.at[pl.ds(cc, 16)], iv, mask=m)
                return c2 + n

            return lax.cond(n > 0, found, lambda c2: c2, c)

        lax.fori_loop(0, 16 * (_CAP // 16), cbody, jnp.int32(0))
        pltpu.sync_copy(ob, ov.at[cid])
        pltpu.sync_copy(ib, oi.at[cid])


def _stage2(r0f, r1f, r2f, thrf, flgf, interpret=False):
    mesh = plsc.VectorSubcoreMesh(core_axis_name="c", subcore_axis_name="s")
    f = pl.kernel(
        _sc_body,
        out_type=[
            jax.ShapeDtypeStruct((2, _CCAP), jnp.float32),
            jax.ShapeDtypeStruct((2, _CCAP), jnp.int32),
        ],
        mesh=mesh,
        scratch_types=[
            pltpu.VMEM((24576,), jnp.float32),
            pltpu.VMEM((_CAP,), jnp.float32),
            pltpu.VMEM((_CAP,), jnp.int32),
            pltpu.VMEM((16,), jnp.float32),
            pltpu.VMEM((2704,), jnp.int32),
            pltpu.VMEM_SHARED((16, _CAP), jnp.float32),
            pltpu.VMEM_SHARED((16, _CAP), jnp.int32),
            pltpu.VMEM((16, _CAP), jnp.float32),
            pltpu.VMEM((16, _CAP), jnp.int32),
            pltpu.VMEM((_CCAP,), jnp.float32),
            pltpu.VMEM((_CCAP,), jnp.int32),
        ],
        compiler_params=pltpu.CompilerParams(needs_layout_passes=False),
        interpret=interpret,
    )
    return f(r0f, r1f, r2f, thrf, flgf)


_SIGLI = tuple(float(s) for s in _SIG[1:4])
_STAB = tuple(tuple(float(np.float32(_SIG[li] * (2.0 ** o) * _MRSIZE))
                    for li in (1, 2, 3)) for o in range(3))


def _k3_body(vcol_ref, icol_ref, vrow_ref, irow_ref,
             oval_ref, os_ref, ocx_ref, ocy_ref):
    vcol = vcol_ref[...]
    icol = icol_ref[...]
    rank_chunks = []
    for jb in range(_NCAND // 128):
        vj = vrow_ref[:, pl.ds(jb * 128, 128)]
        ij = irow_ref[:, pl.ds(jb * 128, 128)]
        beats = (vcol > vj) | ((vcol == vj) & (icol < ij))
        rank_chunks.append(jnp.sum(beats.astype(jnp.int32), axis=0,
                                   keepdims=True))
    ranks = jnp.concatenate(rank_chunks, axis=1)

    vrow = vrow_ref[...]
    irow = irow_ref[...]
    piota = lax.broadcasted_iota(jnp.int32, (512, _NCAND), 0)
    sel = ranks == piota
    val = jnp.sum(jnp.where(sel, vrow, 0.0), axis=1)
    idx = jnp.sum(jnp.where(sel, irow, 0), axis=1)

    is2 = idx >= _SEG_BASES[2]
    is1 = (idx >= _SEG_BASES[1]) & (~is2)
    q = idx - jnp.where(is2, _SEG_BASES[2],
                        jnp.where(is1, _SEG_BASES[1], 0))
    sh_li = jnp.where(is2, 14, jnp.where(is1, 16, 18))
    lix = lax.shift_right_logical(q, sh_li)
    rem = q & (lax.shift_left(jnp.ones_like(q), sh_li) - 1)
    sh_w = jnp.where(is2, 7, jnp.where(is1, 8, 9))
    y = lax.shift_right_logical(rem, sh_w)
    xq = rem & (lax.shift_left(jnp.ones_like(rem), sh_w) - 1)
    scale = jnp.where(is2, 4.0, jnp.where(is1, 2.0, 1.0)).astype(jnp.float32)

    def stab_sel(o):
        return jnp.where(lix == 0, _STAB[o][0],
                         jnp.where(lix == 1, _STAB[o][1], _STAB[o][2]))

    sv = jnp.where(is2, stab_sel(2), jnp.where(is1, stab_sel(1), stab_sel(0)))
    oval_ref[...] = val
    os_ref[...] = sv.astype(jnp.float32)
    ocx_ref[...] = xq.astype(jnp.float32) * scale
    ocy_ref[...] = y.astype(jnp.float32) * scale


def _stage3(vcol, icol, vrow, irow, interpret=False):
    out_shape = [jax.ShapeDtypeStruct((512,), jnp.float32)] * 4
    return pl.pallas_call(_k3_body, out_shape=out_shape,
                          interpret=interpret)(vcol, icol, vrow, irow)


def _run(x, interpret=False):
    x2d = x.reshape(512, 512)
    r0, r1, r2, t, flg = _stage1(x2d, interpret=interpret)
    cv, ci = _stage2(r0.reshape(-1), r1.reshape(-1), r2.reshape(-1),
                     t.reshape(-1), flg.reshape(-1), interpret=interpret)
    vflat = jnp.concatenate([cv.reshape(-1), jnp.zeros((_NSYN,), jnp.float32)])
    iflat = jnp.concatenate([ci.reshape(-1),
                             jnp.arange(_NSYN, dtype=jnp.int32)])
    val, s, cx, cy = _stage3(vflat.reshape(_NCAND, 1),
                             iflat.reshape(_NCAND, 1),
                             vflat.reshape(1, _NCAND),
                             iflat.reshape(1, _NCAND),
                             interpret=interpret)
    top = val[:_NUMF]
    z = jnp.zeros((_NUMF,), jnp.float32)
    row0 = jnp.stack([s[:_NUMF], z, cx[:_NUMF]], axis=1)
    row1 = jnp.stack([z, s[:_NUMF], cy[:_NUMF]], axis=1)
    lafs = jnp.stack([row0, row1], axis=1)
    return top, lafs


def kernel(x):
    return _run(x)


# speedup vs baseline: 173.2936x; 1.0585x over previous
import functools
import math

import ml_dtypes
import numpy as np
import jax
import jax.numpy as jnp
from jax import lax
from jax.experimental import pallas as pl
from jax.experimental.pallas import tpu as pltpu
from jax.experimental.pallas import tpu_sc as plsc

_BORDER = 16
_NUMF = 500
_MRSIZE = 3.0
_NLEVELS = 3
_INIT_SIGMA = 1.6
_NOCT = 3
_SIZES = (512, 256, 128)

_SEG_BASES = (0, 786432, 983040)
_TOTAL = 1032192
_NW = 32
_CAP = 128
_CCAP = 768
_NSYN = 512
_NCAND = 2 * _CCAP + _NSYN


def _sig_list():
    sig = [_INIT_SIGMA]
    sd = []
    for i in range(1, _NLEVELS + 2):
        s_tot = _INIT_SIGMA * (2.0 ** (float(i) / _NLEVELS))
        sd.append(math.sqrt(max(s_tot * s_tot - sig[-1] * sig[-1], 1e-8)))
        sig.append(s_tot)
    return sig, sd

_SIG, _SD = _sig_list()


def _gauss1d_np(sigma):
    r = max(int(math.ceil(3.0 * sigma)), 1)
    x = np.arange(-r, r + 1, dtype=np.float32)
    k = np.exp(-0.5 * (x / sigma) ** 2)
    return (k / k.sum()).astype(np.float32)


def _band_np(S, w):
    r = len(w) // 2
    M = np.zeros((S, S), np.float32)
    for k in range(len(w)):
        off = k - r
        i0 = max(0, -off)
        i1 = min(S, S - off)
        idx = np.arange(i0, i1)
        M[idx, idx + off] = w[k]
    return M


def _down_np(S):
    D = np.zeros((S // 2, S), np.float32)
    D[np.arange(S // 2), 2 * np.arange(S // 2)] = 1.0
    return D


def _build_mats():
    bf = ml_dtypes.bfloat16
    mats = []
    mats.append(_band_np(512, _gauss1d_np(_INIT_SIGMA)).astype(bf))
    for S in _SIZES:
        for i in range(4):
            mats.append(_band_np(S, _gauss1d_np(_SD[i])).astype(bf))
    mats.append(_down_np(512).astype(bf))
    mats.append(_down_np(512).T.copy().astype(bf))
    mats.append(_down_np(256).astype(bf))
    mats.append(_down_np(256).T.copy().astype(bf))
    return mats

_MATS = _build_mats()

_NEG = float("-inf")


def _shift_x(a, d, fill):
    H, W = a.shape
    col = jnp.full((H, 1), fill, a.dtype)
    if d == 1:
        return jnp.concatenate([a[:, 1:], col], axis=1)
    return jnp.concatenate([col, a[:, :-1]], axis=1)


def _shift_y(a, d, fill):
    H, W = a.shape
    row = jnp.full((1, W), fill, a.dtype)
    if d == 1:
        return jnp.concatenate([a[1:, :], row], axis=0)
    return jnp.concatenate([row, a[:-1, :]], axis=0)


def _hessian_resp(l, sigma):
    z = jnp.float32(0.0)
    rn = _shift_x(l, 1, z)
    ln = _shift_x(l, -1, z)
    dn = _shift_y(l, 1, z)
    up = _shift_y(l, -1, z)
    gxx = ln + rn - 2.0 * l
    gyy = up + dn - 2.0 * l
    h = _shift_x(l, -1, z) - _shift_x(l, 1, z)
    gxy = 0.25 * (_shift_y(h, -1, z) - _shift_y(h, 1, z))
    r = jnp.float32(sigma ** 4) * (gxx * gyy - gxy * gxy)
    return jnp.maximum(r, 0.0)


def _smax3(a):
    m = jnp.maximum(a, jnp.maximum(_shift_x(a, 1, _NEG), _shift_x(a, -1, _NEG)))
    return jnp.maximum(m, jnp.maximum(_shift_y(m, 1, _NEG), _shift_y(m, -1, _NEG)))


def _fold_chunks(v):
    H, W = v.shape
    m = v[:, :128]
    for k in range(1, W // 128):
        m = jnp.maximum(m, v[:, k * 128:(k + 1) * 128])
    m = jnp.maximum(m[:H // 2, :], m[H // 2:, :])
    m = jnp.maximum(m[:H // 4, :], m[H // 4:, :])
    return m


def _mm(a, b):
    return jnp.dot(a, b, preferred_element_type=jnp.float32)


def _blur_bf(lv, B):
    y = _mm(B, lv).astype(jnp.bfloat16)
    return _mm(y, B).astype(jnp.bfloat16)


def _k1_body(*refs):
    x = refs[0][...].astype(jnp.bfloat16)
    g_init = refs[1][...]
    gs = {S: [refs[2 + 4 * si + i][...] for i in range(4)]
          for si, S in enumerate(_SIZES)}
    d0, d0t, d1, d1t = (refs[14][...], refs[15][...], refs[16][...],
                        refs[17][...])
    out = {512: refs[18], 256: refs[19], 128: refs[20]}
    tout = refs[21]
    fout = refs[22]

    folds = []
    rowmaxes = []
    cur = _blur_bf(x, g_init)
    for o, S in enumerate(_SIZES):
        levels = [cur]
        for i in range(4):
            levels.append(_blur_bf(levels[-1], gs[S][i]))
        resps = [_hessian_resp(levels[i].astype(jnp.float32), _SIG[i])
                 for i in range(5)]
        smax = [_smax3(r) for r in resps]
        yy = lax.broadcasted_iota(jnp.int32, (S, S), 0)
        xx = lax.broadcasted_iota(jnp.int32, (S, S), 1)
        bm = ((yy >= _BORDER) & (yy < S - _BORDER) &
              (xx >= _BORDER) & (xx < S - _BORDER))
        for li in range(1, 4):
            mx = jnp.maximum(smax[li - 1], jnp.maximum(smax[li], smax[li + 1]))
            ism = (resps[li] >= mx) & (resps[li] > 0.0) & bm
            vals = jnp.where(ism, resps[li], 0.0)
            out[S][li - 1, :, :] = vals
            folds.append(_fold_chunks(vals))
            rowmaxes.append(jnp.max(vals, axis=1, keepdims=True))
        if o == 0:
            cur = _mm(d0, _mm(levels[_NLEVELS], d0t)).astype(jnp.bfloat16)
        elif o == 1:
            cur = _mm(d1, _mm(levels[_NLEVELS], d1t)).astype(jnp.bfloat16)

    cm = jnp.concatenate(folds, axis=0)
    cmi = lax.bitcast_convert_type(cm, jnp.int32)

    def bs(_, lohi):
        lo, hi = lohi
        mid = lo + (hi - lo + 1) // 2
        cnt = jnp.sum((cmi >= mid).astype(jnp.int32))
        ge = cnt >= _NUMF
        return (jnp.where(ge, mid, lo), jnp.where(ge, hi, mid - 1))

    lo, _hi = lax.fori_loop(0, 31, bs, (jnp.int32(0), jnp.int32(2**31 - 2)))
    t0 = lax.bitcast_convert_type(lo, jnp.float32)
    tout[:, :] = jnp.full((8, 128), 1.0, jnp.float32) * t0
    rm = jnp.concatenate(rowmaxes, axis=0)
    fout[:, :] = ((rm >= t0) & (rm > 0.0)).astype(jnp.int32)


def _stage1(x2d, interpret=False):
    out_shape = [
        jax.ShapeDtypeStruct((3, 512, 512), jnp.float32),
        jax.ShapeDtypeStruct((3, 256, 256), jnp.float32),
        jax.ShapeDtypeStruct((3, 128, 128), jnp.float32),
        jax.ShapeDtypeStruct((8, 128), jnp.float32),
        jax.ShapeDtypeStruct((2688, 1), jnp.int32),
    ]
    return pl.pallas_call(_k1_body, out_shape=out_shape,
                          interpret=interpret)(x2d, *_MATS)


_SEGS = ((0, 24576), (786432, 6144), (983040, 1536))


_ROWSEG = ((0, 48, 512, 32, 0), (1536, 24, 256, 16, 786432),
           (2304, 12, 128, 8, 983040))


def _sc_body(r0, r1, r2, thr, flg, ov, oi, buf, cv, ci, tv, fb,
             sh_v, sh_i, lv, li, ob, ib):
    sid = lax.axis_index("s")
    cid = lax.axis_index("c")
    wid = sid * 2 + cid
    pltpu.sync_copy(thr.at[pl.ds(0, 16)], tv)
    pltpu.sync_copy(flg, fb.at[pl.ds(0, 2688)])
    tsplat = tv[...]
    for j in range(_CAP // 16):
        cv[pl.ds(j * 16, 16)] = jnp.full((16,), -1.0, jnp.float32)
        ci[pl.ds(j * 16, 16)] = jnp.full((16,), 1 << 29, jnp.int32)

    refs = (r0, r1, r2)
    cnt = jnp.int32(0)
    for si in range(3):
        fbase, nrows, W, vpr, base = _ROWSEG[si]
        per = nrows * W
        start = wid * per
        pltpu.sync_copy(refs[si].at[pl.ds(start, per)], buf.at[pl.ds(0, per)])

        def rowbody(r, c, fbase=fbase, nrows=nrows, vpr=vpr,
                    base=base, start=start):
            flag = fb[pl.ds(fbase + wid * nrows + r, 16)][0]

            def scan_row(c2):
                def body(j, c3):
                    v = buf[pl.ds((r * vpr + j) * 16, 16)]
                    m = (v >= tsplat) & (v > 0.0)
                    n = plsc.all_reduce_population_count(m)[0]

                    def found(c4):
                        cc = jnp.minimum(c4, _CAP - 16)
                        plsc.store_compressed(cv.at[pl.ds(cc, 16)], v, mask=m)
                        iv = (lax.iota(jnp.int32, 16)
                              + (base + start + (r * vpr + j) * 16))
                        plsc.store_compressed(ci.at[pl.ds(cc, 16)], iv,
                                              mask=m)
                        return c4 + n

                    return lax.cond(n > 0, found, lambda c4: c4, c3)

                return lax.fori_loop(0, vpr, body, c2)

            return lax.cond(flag > 0, scan_row, lambda c2: c2, c)

        cnt = lax.fori_loop(0, nrows, rowbody, cnt)

    pltpu.sync_copy(cv, sh_v.at[sid])
    pltpu.sync_copy(ci, sh_i.at[sid])
    plsc.subcore_barrier()

    @pl.when(sid == 0)
    def _compact():
        pltpu.sync_copy(sh_v, lv)
        pltpu.sync_copy(sh_i, li)
        for j in range(_CCAP // 16):
            ob[pl.ds(j * 16, 16)] = jnp.full((16,), -1.0, jnp.float32)
            ib[pl.ds(j * 16, 16)] = jnp.full((16,), 1 << 29, jnp.int32)

        def cbody(k, c):
            w = k // (_CAP // 16)
            j = k % (_CAP // 16)
            v = lv[w, pl.ds(j * 16, 16)]
            m = v > 0.0
            n = plsc.all_reduce_population_count(m)[0]

            def found(c2):
                cc = jnp.minimum(c2, _CCAP - 16)
                plsc.store_compressed(ob.at[pl.ds(cc, 16)], v, mask=m)
                iv = li[w, pl.ds(j * 16, 16)]
                plsc.store_compressed(ib.at[pl.ds(cc, 16)], iv, mask=m)
                return c2 + n

            return lax.cond(n > 0, found, lambda c2: c2, c)

        lax.fori_loop(0, 16 * (_CAP // 16), cbody, jnp.int32(0))
        pltpu.sync_copy(ob, ov.at[cid])
        pltpu.sync_copy(ib, oi.at[cid])


def _stage2(r0f, r1f, r2f, thrf, flgf, interpret=False):
    mesh = plsc.VectorSubcoreMesh(core_axis_name="c", subcore_axis_name="s")
    f = pl.kernel(
        _sc_body,
        out_type=[
            jax.ShapeDtypeStruct((2, _CCAP), jnp.float32),
            jax.ShapeDtypeStruct((2, _CCAP), jnp.int32),
        ],
        mesh=mesh,
        scratch_types=[
            pltpu.VMEM((24576,), jnp.float32),
            pltpu.VMEM((_CAP,), jnp.float32),
            pltpu.VMEM((_CAP,), jnp.int32),
            pltpu.VMEM((16,), jnp.float32),
            pltpu.VMEM((2704,), jnp.int32),
            pltpu.VMEM_SHARED((16, _CAP), jnp.float32),
            pltpu.VMEM_SHARED((16, _CAP), jnp.int32),
            pltpu.VMEM((16, _CAP), jnp.float32),
            pltpu.VMEM((16, _CAP), jnp.int32),
            pltpu.VMEM((_CCAP,), jnp.float32),
            pltpu.VMEM((_CCAP,), jnp.int32),
        ],
        compiler_params=pltpu.CompilerParams(needs_layout_passes=False),
        interpret=interpret,
    )
    return f(r0f, r1f, r2f, thrf, flgf)


_SIGLI = tuple(float(s) for s in _SIG[1:4])
_STAB = tuple(tuple(float(np.float32(_SIG[li] * (2.0 ** o) * _MRSIZE))
                    for li in (1, 2, 3)) for o in range(3))


def _k3_body(vcol_ref, icol_ref, vrow_ref, irow_ref,
             oval_ref, os_ref, ocx_ref, ocy_ref):
    vcol = vcol_ref[...]
    icol = icol_ref[...]
    rank_chunks = []
    for jb in range(_NCAND // 128):
        vj = vrow_ref[:, pl.ds(jb * 128, 128)]
        ij = irow_ref[:, pl.ds(jb * 128, 128)]
        beats = (vcol > vj) | ((vcol == vj) & (icol < ij))
        rank_chunks.append(jnp.sum(beats.astype(jnp.int32), axis=0,
                                   keepdims=True))
    ranks = jnp.concatenate(rank_chunks, axis=1)

    vrow = vrow_ref[...]
    irow = irow_ref[...]
    piota = lax.broadcasted_iota(jnp.int32, (512, _NCAND), 0)
    sel = ranks == piota
    val = jnp.sum(jnp.where(sel, vrow, 0.0), axis=1)
    idx = jnp.sum(jnp.where(sel, irow, 0), axis=1)

    is2 = idx >= _SEG_BASES[2]
    is1 = (idx >= _SEG_BASES[1]) & (~is2)
    q = idx - jnp.where(is2, _SEG_BASES[2],
                        jnp.where(is1, _SEG_BASES[1], 0))
    sh_li = jnp.where(is2, 14, jnp.where(is1, 16, 18))
    lix = lax.shift_right_logical(q, sh_li)
    rem = q & (lax.shift_left(jnp.ones_like(q), sh_li) - 1)
    sh_w = jnp.where(is2, 7, jnp.where(is1, 8, 9))
    y = lax.shift_right_logical(rem, sh_w)
    xq = rem & (lax.shift_left(jnp.ones_like(rem), sh_w) - 1)
    scale = jnp.where(is2, 4.0, jnp.where(is1, 2.0, 1.0)).astype(jnp.float32)

    def stab_sel(o):
        return jnp.where(lix == 0, _STAB[o][0],
                         jnp.where(lix == 1, _STAB[o][1], _STAB[o][2]))

    sv = jnp.where(is2, stab_sel(2), jnp.where(is1, stab_sel(1), stab_sel(0)))
    oval_ref[...] = val
    os_ref[...] = sv.astype(jnp.float32)
    ocx_ref[...] = xq.astype(jnp.float32) * scale
    ocy_ref[...] = y.astype(jnp.float32) * scale


def _stage3(vcol, icol, vrow, irow, interpret=False):
    out_shape = [jax.ShapeDtypeStruct((512,), jnp.float32)] * 4
    return pl.pallas_call(_k3_body, out_shape=out_shape,
                          interpret=interpret)(vcol, icol, vrow, irow)


def _run(x, interpret=False):
    x2d = x.reshape(512, 512)
    r0, r1, r2, t, flg = _stage1(x2d, interpret=interpret)
    cv, ci = _stage2(r0.reshape(-1), r1.reshape(-1), r2.reshape(-1),
                     t.reshape(-1), flg.reshape(-1), interpret=interpret)
    vflat = jnp.concatenate([cv.reshape(-1), jnp.zeros((_NSYN,), jnp.float32)])
    iflat = jnp.concatenate([ci.reshape(-1),
                             jnp.arange(_NSYN, dtype=jnp.int32)])
    val, s, cx, cy = _stage3(vflat.reshape(_NCAND, 1),
                             iflat.reshape(_NCAND, 1),
                             vflat.reshape(1, _NCAND),
                             iflat.reshape(1, _NCAND),
                             interpret=interpret)
    top = val[:_NUMF]
    z = jnp.zeros((_NUMF,), jnp.float32)
    row0 = jnp.stack([s[:_NUMF], z, cx[:_NUMF]], axis=1)
    row1 = jnp.stack([z, s[:_NUMF], cy[:_NUMF]], axis=1)
    lafs = jnp.stack([row0, row1], axis=1)
    return top, lafs


def kernel(x):
    return _run(x)
